# initial kernel scaffold (unmeasured)
import jax
import jax.numpy as jnp
from jax import lax
from jax.experimental import pallas as pl
from jax.experimental.pallas import tpu as pltpu


def kernel(
    x,
):
    def body(*refs):
        pass

    out_shape = jax.ShapeDtypeStruct(..., jnp.float32)
    return pl.pallas_call(body, out_shape=out_shape)(...)


# baseline (device time: 10913 ns/iter reference)
import jax
import jax.numpy as jnp
from jax import lax
from jax.experimental import pallas as pl
from jax.experimental.pallas import tpu as pltpu

N_DEV = 4


def kernel(x):
    m_per, n = x.shape

    def body(x_ref, out_ref, own_ref, rbuf_ref, send_sems, recv_sems):
        my_pos = lax.axis_index("i")

        vals = x_ref[:, :]
        mx = jnp.max(vals, axis=0, keepdims=True)
        rows = lax.broadcasted_iota(jnp.int32, (m_per, n), 0)
        masked = jnp.where(vals == mx, rows, N_DEV * m_per)
        lidx = jnp.min(masked, axis=0, keepdims=True)
        gidx = (lidx + my_pos * m_per).astype(jnp.float32)
        own_ref[:, :] = jnp.concatenate([mx, gidx], axis=0)

        barrier_sem = pltpu.get_barrier_semaphore()
        for p in range(N_DEV):

            @pl.when(my_pos != p)
            def _():
                pl.semaphore_signal(
                    barrier_sem,
                    inc=1,
                    device_id=(p,),
                    device_id_type=pl.DeviceIdType.MESH,
                )

        pl.semaphore_wait(barrier_sem, N_DEV - 1)

        rdmas = []
        for d in range(1, N_DEV):
            rdma = pltpu.make_async_remote_copy(
                src_ref=own_ref,
                dst_ref=rbuf_ref.at[d - 1],
                send_sem=send_sems.at[d - 1],
                recv_sem=recv_sems.at[d - 1],
                device_id=((my_pos + d) % N_DEV,),
                device_id_type=pl.DeviceIdType.MESH,
            )
            rdma.start()
            rdmas.append(rdma)
        for rdma in rdmas:
            rdma.wait()

        best_v = own_ref[0:1, :]
        best_i = own_ref[1:2, :]
        for k in range(N_DEV - 1):
            v = rbuf_ref[k, 0:1, :]
            i = rbuf_ref[k, 1:2, :]
            take = (v > best_v) | ((v == best_v) & (i < best_i))
            best_v = jnp.where(take, v, best_v)
            best_i = jnp.where(take, i, best_i)
        out_ref[:, :] = jnp.concatenate([best_v, best_i], axis=0)

    return pl.pallas_call(
        body,
        out_shape=jax.ShapeDtypeStruct((2, n), jnp.float32),
        in_specs=[pl.BlockSpec(memory_space=pltpu.VMEM)],
        out_specs=pl.BlockSpec(memory_space=pltpu.VMEM),
        scratch_shapes=[
            pltpu.VMEM((2, n), jnp.float32),
            pltpu.VMEM((N_DEV - 1, 2, n), jnp.float32),
            pltpu.SemaphoreType.DMA((N_DEV - 1,)),
            pltpu.SemaphoreType.DMA((N_DEV - 1,)),
        ],
        compiler_params=pltpu.CompilerParams(collective_id=0),
    )(x)


# device time: 10718 ns/iter; 1.0182x vs baseline; 1.0182x over previous
import jax
import jax.numpy as jnp
from jax import lax
from jax.experimental import pallas as pl
from jax.experimental.pallas import tpu as pltpu

N_DEV = 4
N_CHUNKS = 8


def kernel(x):
    m_per, n = x.shape
    rows_per = m_per // N_CHUNKS

    def body(x_ref, out_ref, acc_ref, rbuf_ref, send_sems, recv_sems):
        g = pl.program_id(0)
        my_pos = lax.axis_index("i")

        vals = x_ref[:, :]
        mc = jnp.max(vals, axis=0, keepdims=True)
        rows = lax.broadcasted_iota(jnp.int32, (rows_per, n), 0) + g * rows_per
        masked = jnp.where(vals == mc, rows, N_DEV * m_per)
        mi = jnp.min(masked, axis=0, keepdims=True).astype(jnp.float32)

        @pl.when(g == 0)
        def _():
            acc_ref[0:1, :] = mc
            acc_ref[1:2, :] = mi

        @pl.when(g > 0)
        def _():
            bv = acc_ref[0:1, :]
            take = mc > bv
            acc_ref[0:1, :] = jnp.where(take, mc, bv)
            acc_ref[1:2, :] = jnp.where(take, mi, acc_ref[1:2, :])

        @pl.when(g == N_CHUNKS - 1)
        def _():
            acc_ref[1:2, :] = (
                acc_ref[1:2, :] + (my_pos * m_per).astype(jnp.float32)
            )

            barrier_sem = pltpu.get_barrier_semaphore()
            for p in range(N_DEV):

                @pl.when(my_pos != p)
                def _():
                    pl.semaphore_signal(
                        barrier_sem,
                        inc=1,
                        device_id=(p,),
                        device_id_type=pl.DeviceIdType.MESH,
                    )

            pl.semaphore_wait(barrier_sem, N_DEV - 1)

            rdmas = []
            for d in range(1, N_DEV):
                rdma = pltpu.make_async_remote_copy(
                    src_ref=acc_ref,
                    dst_ref=rbuf_ref.at[d - 1],
                    send_sem=send_sems.at[d - 1],
                    recv_sem=recv_sems.at[d - 1],
                    device_id=((my_pos + d) % N_DEV,),
                    device_id_type=pl.DeviceIdType.MESH,
                )
                rdma.start()
                rdmas.append(rdma)
            for rdma in rdmas:
                rdma.wait()

            best_v = acc_ref[0:1, :]
            best_i = acc_ref[1:2, :]
            for k in range(N_DEV - 1):
                v = rbuf_ref[k, 0:1, :]
                i = rbuf_ref[k, 1:2, :]
                take = (v > best_v) | ((v == best_v) & (i < best_i))
                best_v = jnp.where(take, v, best_v)
                best_i = jnp.where(take, i, best_i)
            out_ref[0:1, :] = best_v
            out_ref[1:2, :] = best_i

    return pl.pallas_call(
        body,
        grid=(N_CHUNKS,),
        out_shape=jax.ShapeDtypeStruct((2, n), jnp.float32),
        in_specs=[pl.BlockSpec((rows_per, n), lambda g: (g, 0))],
        out_specs=pl.BlockSpec((2, n), lambda g: (0, 0)),
        scratch_shapes=[
            pltpu.VMEM((2, n), jnp.float32),
            pltpu.VMEM((N_DEV - 1, 2, n), jnp.float32),
            pltpu.SemaphoreType.DMA((N_DEV - 1,)),
            pltpu.SemaphoreType.DMA((N_DEV - 1,)),
        ],
        compiler_params=pltpu.CompilerParams(collective_id=0),
    )(x)
